# baseline (speedup 1.0000x reference)
import functools

import jax
import jax.numpy as jnp
from jax import lax
from jax.experimental import pallas as pl
from jax.experimental.pallas import tpu as pltpu
from jax.experimental.pallas import tpu_sc as plsc

_W = 80
_NBUF = 2


def _gather_rows(table, idx_flat):
    B = idx_flat.shape[0]
    V, D = table.shape
    NC, NS = 2, 16
    NW = NC * NS
    PW = B // NW
    G = PW // _W
    assert PW % _W == 0 and G % _NBUF == 0 and B % NW == 0
    mesh = plsc.VectorSubcoreMesh(core_axis_name="c", subcore_axis_name="s")

    @functools.partial(
        pl.kernel,
        mesh=mesh,
        out_type=jax.ShapeDtypeStruct((B, D), table.dtype),
        scratch_types=[
            pltpu.VMEM((PW,), jnp.int32),
            pltpu.VMEM((_NBUF, _W, D), table.dtype),
        ]
        + [pltpu.SemaphoreType.DMA] * (2 * _NBUF),
    )
    def emb(table_hbm, idx_hbm, out_hbm, idx_v, bufs, *sems):
        gsem, wsem = sems[:_NBUF], sems[_NBUF:]
        wid = lax.axis_index("s") * NC + lax.axis_index("c")
        base = wid * PW
        pltpu.sync_copy(idx_hbm.at[pl.ds(base, PW)], idx_v)

        def gather_copy(k, b):
            return pltpu.make_async_copy(
                table_hbm.at[idx_v.at[pl.ds(pl.multiple_of(k * _W, 8), _W)]],
                bufs.at[b],
                gsem[b],
            )

        def write_copy(k, b):
            return pltpu.make_async_copy(
                bufs.at[b], out_hbm.at[pl.ds(base + k * _W, _W)], wsem[b]
            )

        for b in range(_NBUF):
            gather_copy(b, b).start()

        @pl.loop(0, G, step=_NBUF)
        def _(g):
            for b in range(_NBUF):
                k = g + b
                bp = (b - 1) % _NBUF
                kr = k - 1 + _NBUF

                @pl.when(jnp.logical_and(kr >= _NBUF, kr < G))
                def _():
                    write_copy(kr - _NBUF, bp).wait()
                    gather_copy(kr, bp).start()

                gather_copy(k, b).wait()
                write_copy(k, b).start()

        for b in range(_NBUF):
            write_copy(0, b).wait()

    return emb(table, idx_flat)


def kernel(input_ids, table):
    Bb, S = input_ids.shape
    D = table.shape[1]
    out = _gather_rows(table, input_ids.T.reshape(Bb * S))
    return out.reshape(S, Bb, D).transpose(1, 0, 2)


# speedup vs baseline: 1.0153x; 1.0153x over previous
import functools

import jax
import jax.numpy as jnp
from jax import lax
from jax.experimental import pallas as pl
from jax.experimental.pallas import tpu as pltpu
from jax.experimental.pallas import tpu_sc as plsc

_W = 16
_NBUF = 10


def _gather_rows(table, idx_flat):
    B = idx_flat.shape[0]
    V, D = table.shape
    NC, NS = 2, 16
    NW = NC * NS
    PW = B // NW
    G = PW // _W
    assert PW % _W == 0 and G % _NBUF == 0 and B % NW == 0
    mesh = plsc.VectorSubcoreMesh(core_axis_name="c", subcore_axis_name="s")

    @functools.partial(
        pl.kernel,
        mesh=mesh,
        out_type=jax.ShapeDtypeStruct((B, D), table.dtype),
        scratch_types=[
            pltpu.VMEM((PW,), jnp.int32),
            pltpu.VMEM((_NBUF, _W, D), table.dtype),
        ]
        + [pltpu.SemaphoreType.DMA] * (2 * _NBUF),
    )
    def emb(table_hbm, idx_hbm, out_hbm, idx_v, bufs, *sems):
        gsem, wsem = sems[:_NBUF], sems[_NBUF:]
        wid = lax.axis_index("s") * NC + lax.axis_index("c")
        base = wid * PW
        pltpu.sync_copy(idx_hbm.at[pl.ds(base, PW)], idx_v)

        def gather_copy(k, b):
            return pltpu.make_async_copy(
                table_hbm.at[idx_v.at[pl.ds(pl.multiple_of(k * _W, 8), _W)]],
                bufs.at[b],
                gsem[b],
            )

        def write_copy(k, b):
            return pltpu.make_async_copy(
                bufs.at[b], out_hbm.at[pl.ds(base + k * _W, _W)], wsem[b]
            )

        for b in range(_NBUF):
            gather_copy(b, b).start()

        @pl.loop(0, G, step=_NBUF)
        def _(g):
            for b in range(_NBUF):
                k = g + b
                bp = (b - 1) % _NBUF
                kr = k - 1 + _NBUF

                @pl.when(jnp.logical_and(kr >= _NBUF, kr < G))
                def _():
                    write_copy(kr - _NBUF, bp).wait()
                    gather_copy(kr, bp).start()

                gather_copy(k, b).wait()
                write_copy(k, b).start()

        for b in range(_NBUF):
            write_copy(0, b).wait()

    return emb(table, idx_flat)


def kernel(input_ids, table):
    Bb, S = input_ids.shape
    D = table.shape[1]
    out = _gather_rows(table, input_ids.T.reshape(Bb * S))
    return out.reshape(S, Bb, D).transpose(1, 0, 2)
